# Initial kernel scaffold; baseline (speedup 1.0000x reference)
import functools

import jax
import jax.numpy as jnp
from jax import lax
from jax.experimental import pallas as pl
from jax.experimental.pallas import tpu as pltpu
from jax.experimental.pallas import tpu_sc as plsc

N = 10000
D = 128
NC = 2
NS = 16
NW = NC * NS
LANES = 16

CHUNK = 128
N_PAD = 10240


def _dense_proj_kernel(seq_ref, wfc_ref, wgat_ref, a2_ref, h_ref, asd_ref,
                       pmax_ref):
    wc = jnp.dot(wfc_ref[...], wgat_ref[...], preferred_element_type=jnp.float32)
    h = jnp.dot(seq_ref[...], wc, preferred_element_type=jnp.float32)
    asd = jnp.dot(h, a2_ref[...], preferred_element_type=jnp.float32)
    h_ref[...] = h
    asd_ref[...] = asd
    pmax_ref[...] = jnp.max(asd, axis=0, keepdims=True)


def _finish_kernel(acc_ref, den_ref, bsum_ref, pa_ref, out_ref):
    s = acc_ref[0] + acc_ref[1]
    d = jnp.sum(den_ref[...], axis=0)
    o = s / d[:, None] + bsum_ref[...]
    a = pa_ref[0, 0]
    out_ref[...] = jnp.where(o >= 0, o, a * o)


def _sc_edge_kernel(nchunks,
                    src_flat_hbm, dst_flat_hbm, dst2d_hbm, as_hbm, ad_hbm,
                    h_hbm, m_hbm,
                    acc_out, den_out,
                    src_v, dst_v, dst2d_v, as_v, ad_v, den_v, rows_v, w_v,
                    m_v, acc_sh, sem):
    cid = lax.axis_index("c")
    sid = lax.axis_index("s")
    wid = cid * NS + sid
    rows_per_tile = N_PAD // NS

    pltpu.sync_copy(src_flat_hbm.at[wid], src_v)
    pltpu.sync_copy(dst_flat_hbm.at[wid], dst_v)
    pltpu.sync_copy(dst2d_hbm.at[wid], dst2d_v)
    pltpu.sync_copy(as_hbm, as_v)
    pltpu.sync_copy(ad_hbm, ad_v)
    pltpu.sync_copy(m_hbm, m_v)
    mvec = m_v[...]

    def zden(i, _):
        den_v[pl.ds(i * LANES, LANES)] = jnp.zeros((LANES,), jnp.float32)
        return 0
    lax.fori_loop(0, N_PAD // LANES, zden, 0)

    def zrow(i, _):
        for j in range(D // LANES):
            rows_v[i, pl.ds(j * LANES, LANES)] = jnp.zeros((LANES,), jnp.float32)
        return 0
    lax.fori_loop(0, CHUNK, zrow, 0)
    for k in range(rows_per_tile // CHUNK):
        pltpu.sync_copy(rows_v, acc_sh.at[pl.ds(sid * rows_per_tile + k * CHUNK,
                                                CHUNK)])
    plsc.subcore_barrier()

    def chunk_body(c, _):
        base = c * CHUNK
        cp = pltpu.async_copy(h_hbm.at[src_v.at[pl.ds(base, CHUNK)]], rows_v,
                              sem)
        cp.wait()

        def grp(g, _):
            off = base + g * LANES
            sidx = src_v[pl.ds(off, LANES)]
            didx = dst_v[pl.ds(off, LANES)]
            a_s = plsc.load_gather(as_v, [sidx])
            a_d = plsc.load_gather(ad_v, [didx])
            e = a_s + a_d
            e = jnp.where(e > 0, e, 0.2 * e)
            w = jnp.exp(e - mvec)
            plsc.addupdate_scatter(den_v, [didx], w)
            w_v[pl.ds(g * LANES, LANES)] = w
            return 0
        lax.fori_loop(0, CHUNK // LANES, grp, 0)

        def scale(i, _):
            ws = plsc.load_gather(w_v, [jnp.full((LANES,), i, jnp.int32)])
            for j in range(D // LANES):
                sl = pl.ds(j * LANES, LANES)
                rows_v[i, sl] = rows_v[i, sl] * ws
            return 0
        lax.fori_loop(0, CHUNK, scale, 0)

        pltpu.sync_copy(rows_v, acc_sh.at[dst2d_v.at[c]], add=True)
        return 0

    lax.fori_loop(0, nchunks, chunk_body, 0)
    plsc.subcore_barrier()

    stripe = pl.ds(sid * rows_per_tile, rows_per_tile)
    pltpu.sync_copy(acc_sh.at[stripe], acc_out.at[cid, stripe])
    pltpu.sync_copy(den_v, den_out.at[wid])


def kernel(seq, edge_index, W_fc, W_gat, a_src, a_dst, b_conv, bias, prelu_a):
    n = seq.shape[0]
    e = edge_index.shape[1]

    blk = 2000
    grid = n // blk
    a2 = jnp.stack([a_src, a_dst], axis=1)
    h, asd, pmax = pl.pallas_call(
        _dense_proj_kernel,
        grid=(grid,),
        in_specs=[
            pl.BlockSpec((blk, D), lambda i: (i, 0)),
            pl.BlockSpec((D, D), lambda i: (0, 0)),
            pl.BlockSpec((D, D), lambda i: (0, 0)),
            pl.BlockSpec((D, 2), lambda i: (0, 0)),
        ],
        out_specs=[
            pl.BlockSpec((blk, D), lambda i: (i, 0)),
            pl.BlockSpec((blk, 2), lambda i: (i, 0)),
            pl.BlockSpec((1, 2), lambda i: (i, 0)),
        ],
        out_shape=[
            jax.ShapeDtypeStruct((n, D), jnp.float32),
            jax.ShapeDtypeStruct((n, 2), jnp.float32),
            jax.ShapeDtypeStruct((grid, 2), jnp.float32),
        ],
    )(seq, W_fc, W_gat, a2)

    mtot = jnp.max(pmax[:, 0]) + jnp.max(pmax[:, 1])
    mshift = jnp.where(mtot > 0, mtot, 0.2 * mtot)
    mvec = jnp.full((LANES,), mshift, jnp.float32)

    alpha_s = jnp.pad(asd[:, 0], (0, N_PAD - n))
    alpha_d = jnp.pad(asd[:, 1], (0, N_PAD - n))

    loop_idx = jnp.arange(n, dtype=edge_index.dtype)
    src_all = jnp.concatenate([edge_index[0], loop_idx])
    dst_all = jnp.concatenate([edge_index[1], loop_idx])
    e_tot = e + n
    per_tile = -(-e_tot // (NW * CHUNK)) * CHUNK
    e_pad = per_tile * NW
    src_all = jnp.pad(src_all, (0, e_pad - e_tot)).astype(jnp.int32)
    dst_all = jnp.pad(dst_all, (0, e_pad - e_tot),
                      constant_values=n).astype(jnp.int32)
    src_flat = src_all.reshape(NW, per_tile)
    dst_flat = dst_all.reshape(NW, per_tile)
    dst2d = dst_all.reshape(NW, per_tile // CHUNK, CHUNK)
    nchunks = per_tile // CHUNK

    mesh = plsc.VectorSubcoreMesh(core_axis_name="c", subcore_axis_name="s")
    acc, den = pl.kernel(
        functools.partial(_sc_edge_kernel, nchunks),
        out_type=[
            jax.ShapeDtypeStruct((NC, N_PAD, D), jnp.float32),
            jax.ShapeDtypeStruct((NW, N_PAD), jnp.float32),
        ],
        mesh=mesh,
        scratch_types=[
            pltpu.VMEM((per_tile,), jnp.int32),
            pltpu.VMEM((per_tile,), jnp.int32),
            pltpu.VMEM((per_tile // CHUNK, CHUNK), jnp.int32),
            pltpu.VMEM((N_PAD,), jnp.float32),
            pltpu.VMEM((N_PAD,), jnp.float32),
            pltpu.VMEM((N_PAD,), jnp.float32),
            pltpu.VMEM((CHUNK, D), jnp.float32),
            pltpu.VMEM((CHUNK,), jnp.float32),
            pltpu.VMEM((LANES,), jnp.float32),
            pltpu.VMEM_SHARED((N_PAD, D), jnp.float32),
            pltpu.SemaphoreType.DMA,
        ],
    )(src_flat, dst_flat, dst2d, alpha_s, alpha_d, h, mvec)

    bsum = (b_conv + bias).reshape(1, D)
    pa = prelu_a.reshape(1, 1)
    out = pl.pallas_call(
        _finish_kernel,
        grid=(grid,),
        in_specs=[
            pl.BlockSpec((NC, blk, D), lambda i: (0, i, 0)),
            pl.BlockSpec((NW, blk), lambda i: (0, i)),
            pl.BlockSpec((1, D), lambda i: (0, 0)),
            pl.BlockSpec((1, 1), lambda i: (0, 0)),
        ],
        out_specs=pl.BlockSpec((blk, D), lambda i: (i, 0)),
        out_shape=jax.ShapeDtypeStruct((n, D), jnp.float32),
    )(acc, den, bsum, pa)
    return out


# speedup vs baseline: 24.9912x; 24.9912x over previous
import functools

import jax
import jax.numpy as jnp
from jax import lax
from jax.experimental import pallas as pl
from jax.experimental.pallas import tpu as pltpu
from jax.experimental.pallas import tpu_sc as plsc

N = 10000
D = 128
NC = 2
NS = 16
NW = NC * NS
LANES = 16

CHUNK = 128
N_PAD = 10240


def _dense_proj_kernel(seq_ref, wfc_ref, wgat_ref, a2_ref, h_ref, asd_ref,
                       pmax_ref):
    wc = jnp.dot(wfc_ref[...], wgat_ref[...], preferred_element_type=jnp.float32)
    h = jnp.dot(seq_ref[...], wc, preferred_element_type=jnp.float32)
    asd = jnp.dot(h, a2_ref[...], preferred_element_type=jnp.float32)
    h_ref[...] = h
    asd_ref[...] = asd
    m = jnp.max(asd, axis=0, keepdims=True)
    i = pl.program_id(0)

    @pl.when(i == 0)
    def _():
        pmax_ref[...] = m

    @pl.when(i > 0)
    def _():
        pmax_ref[...] = jnp.maximum(pmax_ref[...], m)


def _finish_kernel(acc_ref, den_ref, bsum_ref, pa_ref, out_ref):
    s = acc_ref[0] + acc_ref[1]
    d = jnp.sum(den_ref[...], axis=0)
    o = s / d[:, None] + bsum_ref[...]
    a = pa_ref[0, 0]
    out_ref[...] = jnp.where(o >= 0, o, a * o)


def _sc_edge_kernel(nchunks,
                    ids_hbm, as_hbm, ad_hbm, h_hbm, m_hbm,
                    acc_out, den_out,
                    ids_v, as_v, ad_v, den_v, rows_v, w_v,
                    m_v, acc_sh, sem):
    cid = lax.axis_index("c")
    sid = lax.axis_index("s")
    wid = cid * NS + sid
    rows_per_tile = N_PAD // NS

    pltpu.sync_copy(as_hbm, as_v)
    pltpu.sync_copy(ad_hbm, ad_v)
    pltpu.sync_copy(m_hbm, m_v)
    mvec = m_v[...]

    def zden(i, _):
        den_v[pl.ds(i * LANES, LANES)] = jnp.zeros((LANES,), jnp.float32)
        return 0
    lax.fori_loop(0, N_PAD // LANES, zden, 0)

    def zrow(i, _):
        for j in range(D // LANES):
            rows_v[i, pl.ds(j * LANES, LANES)] = jnp.zeros((LANES,), jnp.float32)
        return 0
    lax.fori_loop(0, CHUNK, zrow, 0)
    for k in range(rows_per_tile // CHUNK):
        pltpu.sync_copy(rows_v, acc_sh.at[pl.ds(sid * rows_per_tile + k * CHUNK,
                                                CHUNK)])
    plsc.subcore_barrier()

    def chunk_body(c, _):
        pltpu.sync_copy(ids_hbm.at[wid, c], ids_v)
        cp = pltpu.async_copy(h_hbm.at[ids_v.at[0]], rows_v, sem)
        cp.wait()

        def grp(g, _):
            sidx = ids_v[0, pl.ds(g * LANES, LANES)]
            didx = ids_v[1, pl.ds(g * LANES, LANES)]
            a_s = plsc.load_gather(as_v, [sidx])
            a_d = plsc.load_gather(ad_v, [didx])
            e = a_s + a_d
            e = jnp.where(e > 0, e, 0.2 * e)
            w = jnp.exp(e - mvec)
            plsc.addupdate_scatter(den_v, [didx], w)
            w_v[pl.ds(g * LANES, LANES)] = w
            return 0
        lax.fori_loop(0, CHUNK // LANES, grp, 0)

        def scale(i, _):
            ws = plsc.load_gather(w_v, [jnp.full((LANES,), i, jnp.int32)])
            for j in range(D // LANES):
                sl = pl.ds(j * LANES, LANES)
                rows_v[i, sl] = rows_v[i, sl] * ws
            return 0
        lax.fori_loop(0, CHUNK, scale, 0)

        pltpu.sync_copy(rows_v, acc_sh.at[ids_v.at[1]], add=True)
        return 0

    lax.fori_loop(0, nchunks, chunk_body, 0)
    plsc.subcore_barrier()

    stripe = pl.ds(sid * rows_per_tile, rows_per_tile)
    pltpu.sync_copy(acc_sh.at[stripe], acc_out.at[cid, stripe])
    pltpu.sync_copy(den_v, den_out.at[wid])


def kernel(seq, edge_index, W_fc, W_gat, a_src, a_dst, b_conv, bias, prelu_a):
    n = seq.shape[0]
    e = edge_index.shape[1]

    blk = 2000
    grid = n // blk
    a2 = jnp.stack([a_src, a_dst], axis=1)
    h, asd, pmax = pl.pallas_call(
        _dense_proj_kernel,
        grid=(grid,),
        in_specs=[
            pl.BlockSpec((blk, D), lambda i: (i, 0)),
            pl.BlockSpec((D, D), lambda i: (0, 0)),
            pl.BlockSpec((D, D), lambda i: (0, 0)),
            pl.BlockSpec((D, 2), lambda i: (0, 0)),
        ],
        out_specs=[
            pl.BlockSpec((blk, D), lambda i: (i, 0)),
            pl.BlockSpec((blk, 2), lambda i: (i, 0)),
            pl.BlockSpec((1, 2), lambda i: (0, 0)),
        ],
        out_shape=[
            jax.ShapeDtypeStruct((n, D), jnp.float32),
            jax.ShapeDtypeStruct((n, 2), jnp.float32),
            jax.ShapeDtypeStruct((1, 2), jnp.float32),
        ],
    )(seq, W_fc, W_gat, a2)

    mtot = pmax[0, 0] + pmax[0, 1]
    mshift = jnp.where(mtot > 0, mtot, 0.2 * mtot)
    mvec = jnp.full((LANES,), mshift, jnp.float32)

    alpha_s = jnp.pad(asd[:, 0], (0, N_PAD - n))
    alpha_d = jnp.pad(asd[:, 1], (0, N_PAD - n))

    loop_idx = jnp.arange(n, dtype=edge_index.dtype)
    src_all = jnp.concatenate([edge_index[0], loop_idx])
    dst_all = jnp.concatenate([edge_index[1], loop_idx])
    e_tot = e + n
    per_tile = -(-e_tot // (NW * CHUNK)) * CHUNK
    e_pad = per_tile * NW
    src_all = jnp.pad(src_all, (0, e_pad - e_tot)).astype(jnp.int32)
    dst_all = jnp.pad(dst_all, (0, e_pad - e_tot),
                      constant_values=n).astype(jnp.int32)
    nchunks = per_tile // CHUNK
    ids = jnp.stack([src_all.reshape(NW, nchunks, CHUNK),
                     dst_all.reshape(NW, nchunks, CHUNK)], axis=2)

    mesh = plsc.VectorSubcoreMesh(core_axis_name="c", subcore_axis_name="s")
    acc, den = pl.kernel(
        functools.partial(_sc_edge_kernel, nchunks),
        out_type=[
            jax.ShapeDtypeStruct((NC, N_PAD, D), jnp.float32),
            jax.ShapeDtypeStruct((NW, N_PAD), jnp.float32),
        ],
        mesh=mesh,
        compiler_params=pltpu.CompilerParams(needs_layout_passes=False),
        scratch_types=[
            pltpu.VMEM((2, CHUNK), jnp.int32),
            pltpu.VMEM((N_PAD,), jnp.float32),
            pltpu.VMEM((N_PAD,), jnp.float32),
            pltpu.VMEM((N_PAD,), jnp.float32),
            pltpu.VMEM((CHUNK, D), jnp.float32),
            pltpu.VMEM((CHUNK,), jnp.float32),
            pltpu.VMEM((LANES,), jnp.float32),
            pltpu.VMEM_SHARED((N_PAD, D), jnp.float32),
            pltpu.SemaphoreType.DMA,
        ],
    )(ids, alpha_s, alpha_d, h, mvec)

    bsum = (b_conv + bias).reshape(1, D)
    pa = prelu_a.reshape(1, 1)
    fblk = 2048
    out = pl.pallas_call(
        _finish_kernel,
        grid=(N_PAD // fblk,),
        in_specs=[
            pl.BlockSpec((NC, fblk, D), lambda i: (0, i, 0)),
            pl.BlockSpec((NW, fblk), lambda i: (0, i)),
            pl.BlockSpec((1, D), lambda i: (0, 0)),
            pl.BlockSpec((1, 1), lambda i: (0, 0)),
        ],
        out_specs=pl.BlockSpec((fblk, D), lambda i: (i, 0)),
        out_shape=jax.ShapeDtypeStruct((N_PAD, D), jnp.float32),
    )(acc, den, bsum, pa)
    return out[:n]


# speedup vs baseline: 29.6425x; 1.1861x over previous
import functools

import jax
import jax.numpy as jnp
from jax import lax
from jax.experimental import pallas as pl
from jax.experimental.pallas import tpu as pltpu
from jax.experimental.pallas import tpu_sc as plsc

N = 10000
D = 128
NC = 2
NS = 16
NW = NC * NS
LANES = 16

CHUNK = 96
N_PAD = 10112


def _dense_proj_kernel(seq_ref, wfc_ref, wgat_ref, a2_ref, h_ref, asd_ref,
                       pmax_ref):
    wc = jnp.dot(wfc_ref[...], wgat_ref[...], preferred_element_type=jnp.float32)
    h = jnp.dot(seq_ref[...], wc, preferred_element_type=jnp.float32)
    asd = jnp.dot(h, a2_ref[...], preferred_element_type=jnp.float32)
    h_ref[...] = h
    asd_ref[...] = asd
    m = jnp.max(asd, axis=0, keepdims=True)
    i = pl.program_id(0)

    @pl.when(i == 0)
    def _():
        pmax_ref[...] = m

    @pl.when(i > 0)
    def _():
        pmax_ref[...] = jnp.maximum(pmax_ref[...], m)


def _finish_kernel(acc_ref, den_ref, bsum_ref, pa_ref, out_ref):
    s = acc_ref[0] + acc_ref[1]
    d = jnp.sum(den_ref[...], axis=0)
    o = s / d[:, None] + bsum_ref[...]
    a = pa_ref[0, 0]
    out_ref[...] = jnp.where(o >= 0, o, a * o)


def _sc_weights_kernel(per_tile,
                       src_hbm, dst_hbm, as_hbm, ad_hbm, m_hbm,
                       w_out, den_out,
                       src_v, dst_v, as_v, ad_v, den_v, w_t, m_v):
    cid = lax.axis_index("c")
    sid = lax.axis_index("s")
    wid = cid * NS + sid

    pltpu.sync_copy(src_hbm.at[wid], src_v)
    pltpu.sync_copy(dst_hbm.at[wid], dst_v)
    pltpu.sync_copy(as_hbm, as_v)
    pltpu.sync_copy(ad_hbm, ad_v)
    pltpu.sync_copy(m_hbm, m_v)
    mvec = m_v[...]

    def zden(i, _):
        den_v[pl.ds(i * LANES, LANES)] = jnp.zeros((LANES,), jnp.float32)
        return 0
    lax.fori_loop(0, N_PAD // LANES, zden, 0)

    def grp(g, _):
        sl = pl.ds(g * LANES, LANES)
        sidx = src_v[sl]
        didx = dst_v[sl]
        a_s = plsc.load_gather(as_v, [sidx])
        a_d = plsc.load_gather(ad_v, [didx])
        e = a_s + a_d
        e = jnp.where(e > 0, e, 0.2 * e)
        w = jnp.exp(e - mvec)
        plsc.addupdate_scatter(den_v, [didx], w)
        w_t[sl] = w
        return 0
    lax.fori_loop(0, per_tile // LANES, grp, 0)

    pltpu.sync_copy(w_t, w_out.at[wid])
    pltpu.sync_copy(den_v, den_out.at[wid])


def _sc_scatter_kernel(nchunks,
                       src_hbm, dst2_hbm, w_hbm, h_hbm,
                       acc_out,
                       src_v, dst2_v, wb0, wb1, rows0, rows1, acc_sh,
                       sem_g0, sem_g1, sem_s0, sem_s1, sem_w0, sem_w1):
    cid = lax.axis_index("c")
    sid = lax.axis_index("s")
    wid = cid * NS + sid
    rows_per_tile = N_PAD // NS

    pltpu.sync_copy(src_hbm.at[wid], src_v)
    pltpu.sync_copy(dst2_hbm.at[wid], dst2_v)

    def zrow(i, _):
        for j in range(D // LANES):
            rows0[i, pl.ds(j * LANES, LANES)] = jnp.zeros((LANES,), jnp.float32)
        return 0
    lax.fori_loop(0, CHUNK, zrow, 0)
    base0 = sid * rows_per_tile
    nfull = rows_per_tile // CHUNK
    for k in range(nfull):
        pltpu.sync_copy(rows0, acc_sh.at[pl.ds(base0 + k * CHUNK, CHUNK)])
    rem = rows_per_tile - nfull * CHUNK
    if rem:
        pltpu.sync_copy(rows0.at[pl.ds(0, rem)],
                        acc_sh.at[pl.ds(base0 + nfull * CHUNK, rem)])
    plsc.subcore_barrier()

    def gather(c, rows, sem):
        off = pl.multiple_of(c * CHUNK, 8)
        return pltpu.async_copy(h_hbm.at[src_v.at[pl.ds(off, CHUNK)]],
                                rows, sem)

    def wfetch(c, wb, sem):
        return pltpu.async_copy(w_hbm.at[wid, c], wb, sem)

    def scale(rows, wb):
        def body(i, _):
            ws = plsc.load_gather(wb, [jnp.full((LANES,), i, jnp.int32)])
            for j in range(D // LANES):
                sl = pl.ds(j * LANES, LANES)
                rows[i, sl] = rows[i, sl] * ws
            return 0
        lax.fori_loop(0, CHUNK, body, 0)

    wfetch(0, wb0, sem_w0)
    wfetch(1, wb1, sem_w1)
    gather(0, rows0, sem_g0)
    gather(1, rows1, sem_g1)

    def step(it, _):
        i = it * 2
        pltpu.make_async_copy(w_hbm.at[wid, i], wb0, sem_w0).wait()
        off0 = pl.multiple_of(i * CHUNK, 8)
        pltpu.make_async_copy(h_hbm.at[src_v.at[pl.ds(off0, CHUNK)]],
                              rows0, sem_g0).wait()
        scale(rows0, wb0)
        cp_s0 = pltpu.async_copy(rows0, acc_sh.at[dst2_v.at[i]], sem_s0,
                                 add=True)

        @pl.when(i + 2 < nchunks)
        def _():
            wfetch(i + 2, wb0, sem_w0)
        pltpu.make_async_copy(w_hbm.at[wid, i + 1], wb1, sem_w1).wait()
        off1 = pl.multiple_of((i + 1) * CHUNK, 8)
        pltpu.make_async_copy(h_hbm.at[src_v.at[pl.ds(off1, CHUNK)]],
                              rows1, sem_g1).wait()
        scale(rows1, wb1)
        cp_s1 = pltpu.async_copy(rows1, acc_sh.at[dst2_v.at[i + 1]], sem_s1,
                                 add=True)

        @pl.when(i + 3 < nchunks)
        def _():
            wfetch(i + 3, wb1, sem_w1)
        cp_s0.wait()

        @pl.when(i + 2 < nchunks)
        def _():
            gather(i + 2, rows0, sem_g0)
        cp_s1.wait()

        @pl.when(i + 3 < nchunks)
        def _():
            gather(i + 3, rows1, sem_g1)
        return 0

    lax.fori_loop(0, nchunks // 2, step, 0)
    plsc.subcore_barrier()

    stripe = pl.ds(sid * rows_per_tile, rows_per_tile)
    pltpu.sync_copy(acc_sh.at[stripe], acc_out.at[cid, stripe])


def kernel(seq, edge_index, W_fc, W_gat, a_src, a_dst, b_conv, bias, prelu_a):
    n = seq.shape[0]
    e = edge_index.shape[1]

    blk = 2000
    grid = n // blk
    a2 = jnp.stack([a_src, a_dst], axis=1)
    h, asd, pmax = pl.pallas_call(
        _dense_proj_kernel,
        grid=(grid,),
        in_specs=[
            pl.BlockSpec((blk, D), lambda i: (i, 0)),
            pl.BlockSpec((D, D), lambda i: (0, 0)),
            pl.BlockSpec((D, D), lambda i: (0, 0)),
            pl.BlockSpec((D, 2), lambda i: (0, 0)),
        ],
        out_specs=[
            pl.BlockSpec((blk, D), lambda i: (i, 0)),
            pl.BlockSpec((blk, 2), lambda i: (i, 0)),
            pl.BlockSpec((1, 2), lambda i: (0, 0)),
        ],
        out_shape=[
            jax.ShapeDtypeStruct((n, D), jnp.float32),
            jax.ShapeDtypeStruct((n, 2), jnp.float32),
            jax.ShapeDtypeStruct((1, 2), jnp.float32),
        ],
    )(seq, W_fc, W_gat, a2)

    mtot = pmax[0, 0] + pmax[0, 1]
    mshift = jnp.where(mtot > 0, mtot, 0.2 * mtot)
    mvec = jnp.full((LANES,), mshift, jnp.float32)

    alpha_s = jnp.pad(asd[:, 0], (0, N_PAD - n))
    alpha_d = jnp.pad(asd[:, 1], (0, N_PAD - n))

    loop_idx = jnp.arange(n, dtype=edge_index.dtype)
    src_all = jnp.concatenate([edge_index[0], loop_idx])
    dst_all = jnp.concatenate([edge_index[1], loop_idx])
    e_tot = e + n
    per_tile = -(-e_tot // (NW * CHUNK)) * CHUNK
    e_pad = per_tile * NW
    src_all = jnp.pad(src_all, (0, e_pad - e_tot)).astype(jnp.int32)
    dst_all = jnp.pad(dst_all, (0, e_pad - e_tot),
                      constant_values=n).astype(jnp.int32)
    nchunks = per_tile // CHUNK
    src_flat = src_all.reshape(NW, per_tile)
    dst_flat = dst_all.reshape(NW, per_tile)
    dst2 = dst_all.reshape(NW, nchunks, CHUNK)

    mesh = plsc.VectorSubcoreMesh(core_axis_name="c", subcore_axis_name="s")
    w_all, den = pl.kernel(
        functools.partial(_sc_weights_kernel, per_tile),
        out_type=[
            jax.ShapeDtypeStruct((NW, per_tile), jnp.float32),
            jax.ShapeDtypeStruct((NW, N_PAD), jnp.float32),
        ],
        mesh=mesh,
        compiler_params=pltpu.CompilerParams(needs_layout_passes=False),
        scratch_types=[
            pltpu.VMEM((per_tile,), jnp.int32),
            pltpu.VMEM((per_tile,), jnp.int32),
            pltpu.VMEM((N_PAD,), jnp.float32),
            pltpu.VMEM((N_PAD,), jnp.float32),
            pltpu.VMEM((N_PAD,), jnp.float32),
            pltpu.VMEM((per_tile,), jnp.float32),
            pltpu.VMEM((LANES,), jnp.float32),
        ],
    )(src_flat, dst_flat, alpha_s, alpha_d, mvec)

    acc = pl.kernel(
        functools.partial(_sc_scatter_kernel, nchunks),
        out_type=jax.ShapeDtypeStruct((NC, N_PAD, D), jnp.float32),
        mesh=mesh,
        compiler_params=pltpu.CompilerParams(needs_layout_passes=False),
        scratch_types=[
            pltpu.VMEM((per_tile,), jnp.int32),
            pltpu.VMEM((nchunks, CHUNK), jnp.int32),
            pltpu.VMEM((CHUNK,), jnp.float32),
            pltpu.VMEM((CHUNK,), jnp.float32),
            pltpu.VMEM((CHUNK, D), jnp.float32),
            pltpu.VMEM((CHUNK, D), jnp.float32),
            pltpu.VMEM_SHARED((N_PAD, D), jnp.float32),
            pltpu.SemaphoreType.DMA,
            pltpu.SemaphoreType.DMA,
            pltpu.SemaphoreType.DMA,
            pltpu.SemaphoreType.DMA,
            pltpu.SemaphoreType.DMA,
            pltpu.SemaphoreType.DMA,
        ],
    )(src_flat, dst2, w_all.reshape(NW, nchunks, CHUNK), h)

    bsum = (b_conv + bias).reshape(1, D)
    pa = prelu_a.reshape(1, 1)
    fblk = 128
    out = pl.pallas_call(
        _finish_kernel,
        grid=(N_PAD // fblk,),
        in_specs=[
            pl.BlockSpec((NC, fblk, D), lambda i: (0, i, 0)),
            pl.BlockSpec((NW, fblk), lambda i: (0, i)),
            pl.BlockSpec((1, D), lambda i: (0, 0)),
            pl.BlockSpec((1, 1), lambda i: (0, 0)),
        ],
        out_specs=pl.BlockSpec((fblk, D), lambda i: (i, 0)),
        out_shape=jax.ShapeDtypeStruct((N_PAD, D), jnp.float32),
    )(acc, den, bsum, pa)
    return out[:n]


# speedup vs baseline: 30.8086x; 1.0393x over previous
import functools

import jax
import jax.numpy as jnp
from jax import lax
from jax.experimental import pallas as pl
from jax.experimental.pallas import tpu as pltpu
from jax.experimental.pallas import tpu_sc as plsc

N = 10000
D = 128
NC = 2
NS = 16
NW = NC * NS
LANES = 16

CHUNK = 96
N_PAD = 10112


def _dense_proj_kernel(seq_ref, wfc_ref, wgat_ref, a2_ref, h_ref, asd_ref,
                       pmax_ref):
    wc = jnp.dot(wfc_ref[...], wgat_ref[...], preferred_element_type=jnp.float32)
    h = jnp.dot(seq_ref[...], wc, preferred_element_type=jnp.float32)
    asd = jnp.dot(h, a2_ref[...], preferred_element_type=jnp.float32)
    h_ref[...] = h
    asd_ref[...] = asd
    m = jnp.max(asd, axis=0, keepdims=True)
    i = pl.program_id(0)

    @pl.when(i == 0)
    def _():
        pmax_ref[...] = m

    @pl.when(i > 0)
    def _():
        pmax_ref[...] = jnp.maximum(pmax_ref[...], m)


def _finish_kernel(acc_ref, den_ref, bsum_ref, pa_ref, out_ref):
    s = acc_ref[0] + acc_ref[1]
    d = jnp.sum(den_ref[...], axis=0)
    o = s / d[:, None] + bsum_ref[...]
    a = pa_ref[0, 0]
    out_ref[...] = jnp.where(o >= 0, o, a * o)


def _sc_weights_kernel(per_tile,
                       src_hbm, dst_hbm, as_hbm, ad_hbm, m_hbm,
                       w_out, den_out,
                       src_v, dst_v, as_v, ad_v, den_v, w_t, m_v):
    cid = lax.axis_index("c")
    sid = lax.axis_index("s")
    wid = cid * NS + sid

    pltpu.sync_copy(src_hbm.at[wid], src_v)
    pltpu.sync_copy(dst_hbm.at[wid], dst_v)
    pltpu.sync_copy(as_hbm, as_v)
    pltpu.sync_copy(ad_hbm, ad_v)
    pltpu.sync_copy(m_hbm, m_v)
    mvec = m_v[...]

    def zden(i, _):
        den_v[pl.ds(i * LANES, LANES)] = jnp.zeros((LANES,), jnp.float32)
        return 0
    lax.fori_loop(0, N_PAD // LANES, zden, 0)

    def grp(g, _):
        sl = pl.ds(g * LANES, LANES)
        sidx = src_v[sl]
        didx = dst_v[sl]
        a_s = plsc.load_gather(as_v, [sidx])
        a_d = plsc.load_gather(ad_v, [didx])
        e = a_s + a_d
        e = jnp.where(e > 0, e, 0.2 * e)
        w = jnp.exp(e - mvec)
        plsc.addupdate_scatter(den_v, [didx], w)
        w_t[sl] = w
        return 0
    lax.fori_loop(0, per_tile // LANES, grp, 0)

    pltpu.sync_copy(w_t, w_out.at[wid])
    pltpu.sync_copy(den_v, den_out.at[wid])


def _sc_scatter_kernel(nchunks,
                       src_hbm, dst2_hbm, w_hbm, h_hbm,
                       acc_out,
                       src_v, dst2_v, wb0, wb1, rows0, rows1, acc_sh,
                       sem_g0, sem_g1, sem_s0, sem_s1, sem_w0, sem_w1):
    cid = lax.axis_index("c")
    sid = lax.axis_index("s")
    wid = cid * NS + sid
    rows_per_tile = N_PAD // NS

    pltpu.sync_copy(src_hbm.at[wid], src_v)
    pltpu.sync_copy(dst2_hbm.at[wid], dst2_v)

    def zrow(i, _):
        for j in range(D // LANES):
            rows0[i, pl.ds(j * LANES, LANES)] = jnp.zeros((LANES,), jnp.float32)
        return 0
    lax.fori_loop(0, CHUNK, zrow, 0)
    base0 = sid * rows_per_tile
    nfull = rows_per_tile // CHUNK
    for k in range(nfull):
        pltpu.sync_copy(rows0, acc_sh.at[pl.ds(base0 + k * CHUNK, CHUNK)])
    rem = rows_per_tile - nfull * CHUNK
    if rem:
        pltpu.sync_copy(rows0.at[pl.ds(0, rem)],
                        acc_sh.at[pl.ds(base0 + nfull * CHUNK, rem)])
    plsc.subcore_barrier()

    def gather(c, rows, sem):
        off = pl.multiple_of(c * CHUNK, 8)
        return pltpu.async_copy(h_hbm.at[src_v.at[pl.ds(off, CHUNK)]],
                                rows, sem)

    def wfetch(c, wb, sem):
        return pltpu.async_copy(w_hbm.at[wid, c], wb, sem)

    def scale(rows, wb):
        def body(i, _):
            ws = plsc.load_gather(wb, [jnp.full((LANES,), i, jnp.int32)])
            for j in range(D // LANES):
                sl = pl.ds(j * LANES, LANES)
                rows[i, sl] = rows[i, sl] * ws
            return 0
        lax.fori_loop(0, CHUNK, body, 0)

    wfetch(0, wb0, sem_w0)
    wfetch(1, wb1, sem_w1)
    gather(0, rows0, sem_g0)
    gather(1, rows1, sem_g1)

    def step(it, _):
        i = it * 2
        pltpu.make_async_copy(w_hbm.at[wid, i], wb0, sem_w0).wait()
        off0 = pl.multiple_of(i * CHUNK, 8)
        pltpu.make_async_copy(h_hbm.at[src_v.at[pl.ds(off0, CHUNK)]],
                              rows0, sem_g0).wait()
        scale(rows0, wb0)
        cp_s0 = pltpu.async_copy(rows0, acc_sh.at[dst2_v.at[i]], sem_s0,
                                 add=True)

        @pl.when(i + 2 < nchunks)
        def _():
            wfetch(i + 2, wb0, sem_w0)
        cp_s0.wait()

        @pl.when(i + 2 < nchunks)
        def _():
            gather(i + 2, rows0, sem_g0)
        pltpu.make_async_copy(w_hbm.at[wid, i + 1], wb1, sem_w1).wait()
        off1 = pl.multiple_of((i + 1) * CHUNK, 8)
        pltpu.make_async_copy(h_hbm.at[src_v.at[pl.ds(off1, CHUNK)]],
                              rows1, sem_g1).wait()
        scale(rows1, wb1)
        cp_s1 = pltpu.async_copy(rows1, acc_sh.at[dst2_v.at[i + 1]], sem_s1,
                                 add=True)

        @pl.when(i + 3 < nchunks)
        def _():
            wfetch(i + 3, wb1, sem_w1)
        cp_s1.wait()

        @pl.when(i + 3 < nchunks)
        def _():
            gather(i + 3, rows1, sem_g1)
        return 0

    lax.fori_loop(0, nchunks // 2, step, 0)
    plsc.subcore_barrier()

    stripe = pl.ds(sid * rows_per_tile, rows_per_tile)
    pltpu.sync_copy(acc_sh.at[stripe], acc_out.at[cid, stripe])


def kernel(seq, edge_index, W_fc, W_gat, a_src, a_dst, b_conv, bias, prelu_a):
    n = seq.shape[0]
    e = edge_index.shape[1]

    blk = 2000
    grid = n // blk
    a2 = jnp.stack([a_src, a_dst], axis=1)
    h, asd, pmax = pl.pallas_call(
        _dense_proj_kernel,
        grid=(grid,),
        in_specs=[
            pl.BlockSpec((blk, D), lambda i: (i, 0)),
            pl.BlockSpec((D, D), lambda i: (0, 0)),
            pl.BlockSpec((D, D), lambda i: (0, 0)),
            pl.BlockSpec((D, 2), lambda i: (0, 0)),
        ],
        out_specs=[
            pl.BlockSpec((blk, D), lambda i: (i, 0)),
            pl.BlockSpec((blk, 2), lambda i: (i, 0)),
            pl.BlockSpec((1, 2), lambda i: (0, 0)),
        ],
        out_shape=[
            jax.ShapeDtypeStruct((n, D), jnp.float32),
            jax.ShapeDtypeStruct((n, 2), jnp.float32),
            jax.ShapeDtypeStruct((1, 2), jnp.float32),
        ],
    )(seq, W_fc, W_gat, a2)

    mtot = pmax[0, 0] + pmax[0, 1]
    mshift = jnp.where(mtot > 0, mtot, 0.2 * mtot)
    mvec = jnp.full((LANES,), mshift, jnp.float32)

    alpha_s = jnp.pad(asd[:, 0], (0, N_PAD - n))
    alpha_d = jnp.pad(asd[:, 1], (0, N_PAD - n))

    loop_idx = jnp.arange(n, dtype=edge_index.dtype)
    src_all = jnp.concatenate([edge_index[0], loop_idx])
    dst_all = jnp.concatenate([edge_index[1], loop_idx])
    e_tot = e + n
    per_tile = -(-e_tot // (NW * CHUNK)) * CHUNK
    e_pad = per_tile * NW
    src_all = jnp.pad(src_all, (0, e_pad - e_tot)).astype(jnp.int32)
    dst_all = jnp.pad(dst_all, (0, e_pad - e_tot),
                      constant_values=n).astype(jnp.int32)
    nchunks = per_tile // CHUNK
    src_flat = src_all.reshape(NW, per_tile)
    dst_flat = dst_all.reshape(NW, per_tile)
    dst2 = dst_all.reshape(NW, nchunks, CHUNK)

    mesh = plsc.VectorSubcoreMesh(core_axis_name="c", subcore_axis_name="s")
    w_all, den = pl.kernel(
        functools.partial(_sc_weights_kernel, per_tile),
        out_type=[
            jax.ShapeDtypeStruct((NW, per_tile), jnp.float32),
            jax.ShapeDtypeStruct((NW, N_PAD), jnp.float32),
        ],
        mesh=mesh,
        compiler_params=pltpu.CompilerParams(needs_layout_passes=False),
        scratch_types=[
            pltpu.VMEM((per_tile,), jnp.int32),
            pltpu.VMEM((per_tile,), jnp.int32),
            pltpu.VMEM((N_PAD,), jnp.float32),
            pltpu.VMEM((N_PAD,), jnp.float32),
            pltpu.VMEM((N_PAD,), jnp.float32),
            pltpu.VMEM((per_tile,), jnp.float32),
            pltpu.VMEM((LANES,), jnp.float32),
        ],
    )(src_flat, dst_flat, alpha_s, alpha_d, mvec)

    acc = pl.kernel(
        functools.partial(_sc_scatter_kernel, nchunks),
        out_type=jax.ShapeDtypeStruct((NC, N_PAD, D), jnp.float32),
        mesh=mesh,
        compiler_params=pltpu.CompilerParams(needs_layout_passes=False),
        scratch_types=[
            pltpu.VMEM((per_tile,), jnp.int32),
            pltpu.VMEM((nchunks, CHUNK), jnp.int32),
            pltpu.VMEM((CHUNK,), jnp.float32),
            pltpu.VMEM((CHUNK,), jnp.float32),
            pltpu.VMEM((CHUNK, D), jnp.float32),
            pltpu.VMEM((CHUNK, D), jnp.float32),
            pltpu.VMEM_SHARED((N_PAD, D), jnp.float32),
            pltpu.SemaphoreType.DMA,
            pltpu.SemaphoreType.DMA,
            pltpu.SemaphoreType.DMA,
            pltpu.SemaphoreType.DMA,
            pltpu.SemaphoreType.DMA,
            pltpu.SemaphoreType.DMA,
        ],
    )(src_flat, dst2, w_all.reshape(NW, nchunks, CHUNK), h)

    bsum = (b_conv + bias).reshape(1, D)
    pa = prelu_a.reshape(1, 1)
    fblk = 128
    out = pl.pallas_call(
        _finish_kernel,
        grid=(N_PAD // fblk,),
        in_specs=[
            pl.BlockSpec((NC, fblk, D), lambda i: (0, i, 0)),
            pl.BlockSpec((NW, fblk), lambda i: (0, i)),
            pl.BlockSpec((1, D), lambda i: (0, 0)),
            pl.BlockSpec((1, 1), lambda i: (0, 0)),
        ],
        out_specs=pl.BlockSpec((fblk, D), lambda i: (i, 0)),
        out_shape=jax.ShapeDtypeStruct((N_PAD, D), jnp.float32),
    )(acc, den, bsum, pa)
    return out[:n]


# speedup vs baseline: 32.7389x; 1.0627x over previous
import functools

import jax
import jax.numpy as jnp
from jax import lax
from jax.experimental import pallas as pl
from jax.experimental.pallas import tpu as pltpu
from jax.experimental.pallas import tpu_sc as plsc

N = 10000
D = 128
NC = 2
NS = 16
NW = NC * NS
LANES = 16

CHUNK = 64
NBUF = 3
N_PAD = 10112


def _dense_proj_kernel(seq_ref, wfc_ref, wgat_ref, a2_ref, h_ref, asd_ref,
                       pmax_ref):
    wc = jnp.dot(wfc_ref[...], wgat_ref[...], preferred_element_type=jnp.float32)
    h = jnp.dot(seq_ref[...], wc, preferred_element_type=jnp.float32)
    asd = jnp.dot(h, a2_ref[...], preferred_element_type=jnp.float32)
    h_ref[...] = h
    asd_ref[...] = asd
    m = jnp.max(asd, axis=0, keepdims=True)
    i = pl.program_id(0)

    @pl.when(i == 0)
    def _():
        pmax_ref[...] = m

    @pl.when(i > 0)
    def _():
        pmax_ref[...] = jnp.maximum(pmax_ref[...], m)


def _finish_kernel(acc_ref, den_ref, bsum_ref, pa_ref, out_ref):
    s = acc_ref[0] + acc_ref[1]
    d = jnp.sum(den_ref[...], axis=0)
    o = s / d[:, None] + bsum_ref[...]
    a = pa_ref[0, 0]
    out_ref[...] = jnp.where(o >= 0, o, a * o)


def _sc_weights_kernel(nchunks,
                       src_hbm, dst_hbm, as_hbm, ad_hbm, m_hbm,
                       rec_out, den_out,
                       src_v, dst_v, as_v, ad_v, den_v, rec_t, m_v):
    cid = lax.axis_index("c")
    sid = lax.axis_index("s")
    wid = cid * NS + sid

    pltpu.sync_copy(src_hbm.at[wid], src_v)
    pltpu.sync_copy(dst_hbm.at[wid], dst_v)
    pltpu.sync_copy(as_hbm, as_v)
    pltpu.sync_copy(ad_hbm, ad_v)
    pltpu.sync_copy(m_hbm, m_v)
    mvec = m_v[...]

    def zden(i, _):
        den_v[pl.ds(i * LANES, LANES)] = jnp.zeros((LANES,), jnp.float32)
        return 0
    lax.fori_loop(0, N_PAD // LANES, zden, 0)

    def chunk(c, _):
        ebase = c * CHUNK
        rbase = c * (2 * CHUNK)
        for pos in range(CHUNK // LANES):
            sl = pl.ds(ebase + pos * LANES, LANES)
            sidx = src_v[sl]
            didx = dst_v[sl]
            a_s = plsc.load_gather(as_v, [sidx])
            a_d = plsc.load_gather(ad_v, [didx])
            e = a_s + a_d
            e = jnp.where(e > 0, e, 0.2 * e)
            w = jnp.exp(e - mvec)
            plsc.addupdate_scatter(den_v, [didx], w)
            rec_t[pl.ds(rbase + pos * LANES, LANES)] = sidx
            rec_t[pl.ds(rbase + CHUNK + pos * LANES, LANES)] = (
                plsc.bitcast(w, jnp.int32))
        return 0
    lax.fori_loop(0, nchunks, chunk, 0)

    pltpu.sync_copy(rec_t, rec_out.at[wid])
    pltpu.sync_copy(den_v, den_out.at[wid])


def _sc_scatter_kernel(nchunks,
                       rec_hbm, dst2_hbm, h_hbm,
                       acc_out,
                       dst2_v, pw, rows, acc_sh, sem_f, sem_g, sem_s):
    cid = lax.axis_index("c")
    sid = lax.axis_index("s")
    wid = cid * NS + sid
    rows_per_tile = N_PAD // NS

    pltpu.sync_copy(dst2_hbm.at[wid], dst2_v)

    def zrow(i, _):
        for j in range(D // LANES):
            rows[0][i, pl.ds(j * LANES, LANES)] = jnp.zeros((LANES,),
                                                            jnp.float32)
        return 0
    lax.fori_loop(0, CHUNK, zrow, 0)
    base0 = sid * rows_per_tile
    nfull = rows_per_tile // CHUNK
    zcps = []
    for k in range(nfull):
        zcps.append(pltpu.async_copy(
            rows[0], acc_sh.at[pl.ds(base0 + k * CHUNK, CHUNK)], sem_s[0]))
    rem = rows_per_tile - nfull * CHUNK
    if rem:
        zcps.append(pltpu.async_copy(
            rows[0].at[pl.ds(0, rem)],
            acc_sh.at[pl.ds(base0 + nfull * CHUNK, rem)], sem_s[0]))
    for cp in zcps:
        cp.wait()
    plsc.subcore_barrier()

    def pwfetch(c, b):
        return pltpu.async_copy(rec_hbm.at[wid, c], pw[b], sem_f[b])

    def gather(c, b):
        return pltpu.async_copy(h_hbm.at[pw[b].at[0]], rows[b], sem_g[b])

    def scale(b):
        def body(i, _):
            wsi = plsc.load_gather(pw[b], [jnp.full((LANES,), 1, jnp.int32),
                                           jnp.full((LANES,), i, jnp.int32)])
            ws = plsc.bitcast(wsi, jnp.float32)
            for j in range(D // LANES):
                sl = pl.ds(j * LANES, LANES)
                rows[b][i, sl] = rows[b][i, sl] * ws
            return 0
        lax.fori_loop(0, CHUNK, body, 0)

    for b in range(NBUF):
        pwfetch(b, b)
    pltpu.make_async_copy(rec_hbm.at[wid, 0], pw[0], sem_f[0]).wait()
    gather(0, 0)
    pltpu.make_async_copy(rec_hbm.at[wid, 1], pw[1], sem_f[1]).wait()
    gather(1, 1)

    def step(it, _):
        for k in range(NBUF):
            c = it * NBUF + k
            kp = (k + NBUF - 1) % NBUF
            kn = (k + 2) % NBUF
            pltpu.make_async_copy(h_hbm.at[pw[k].at[0]], rows[k],
                                  sem_g[k]).wait()
            scale(k)
            cp_s = pltpu.async_copy(rows[k], acc_sh.at[dst2_v.at[c]],
                                    sem_s[k], add=True)

            @pl.when(c + NBUF < nchunks)
            def _():
                pwfetch(c + NBUF, k)

            @pl.when(c > 0)
            def _():
                pltpu.make_async_copy(
                    rows[kp], acc_sh.at[dst2_v.at[c - 1]], sem_s[kp]).wait()

            @pl.when(c + 2 < nchunks)
            def _():
                pltpu.make_async_copy(rec_hbm.at[wid, c + 2], pw[kn],
                                      sem_f[kn]).wait()
                gather(c + 2, kn)
            return_val = cp_s
        return 0

    lax.fori_loop(0, nchunks // NBUF, step, 0)
    pltpu.make_async_copy(rows[(nchunks - 1) % NBUF],
                          acc_sh.at[dst2_v.at[nchunks - 1]],
                          sem_s[(nchunks - 1) % NBUF]).wait()
    plsc.subcore_barrier()

    stripe = pl.ds(sid * rows_per_tile, rows_per_tile)
    pltpu.sync_copy(acc_sh.at[stripe], acc_out.at[cid, stripe])


def kernel(seq, edge_index, W_fc, W_gat, a_src, a_dst, b_conv, bias, prelu_a):
    n = seq.shape[0]
    e = edge_index.shape[1]

    blk = 2000
    grid = n // blk
    a2 = jnp.stack([a_src, a_dst], axis=1)
    h, asd, pmax = pl.pallas_call(
        _dense_proj_kernel,
        grid=(grid,),
        in_specs=[
            pl.BlockSpec((blk, D), lambda i: (i, 0)),
            pl.BlockSpec((D, D), lambda i: (0, 0)),
            pl.BlockSpec((D, D), lambda i: (0, 0)),
            pl.BlockSpec((D, 2), lambda i: (0, 0)),
        ],
        out_specs=[
            pl.BlockSpec((blk, D), lambda i: (i, 0)),
            pl.BlockSpec((blk, 2), lambda i: (i, 0)),
            pl.BlockSpec((1, 2), lambda i: (0, 0)),
        ],
        out_shape=[
            jax.ShapeDtypeStruct((n, D), jnp.float32),
            jax.ShapeDtypeStruct((n, 2), jnp.float32),
            jax.ShapeDtypeStruct((1, 2), jnp.float32),
        ],
    )(seq, W_fc, W_gat, a2)

    mtot = pmax[0, 0] + pmax[0, 1]
    mshift = jnp.where(mtot > 0, mtot, 0.2 * mtot)
    mvec = jnp.full((LANES,), mshift, jnp.float32)

    alpha_s = jnp.pad(asd[:, 0], (0, N_PAD - n))
    alpha_d = jnp.pad(asd[:, 1], (0, N_PAD - n))

    loop_idx = jnp.arange(n, dtype=edge_index.dtype)
    src_all = jnp.concatenate([edge_index[0], loop_idx])
    dst_all = jnp.concatenate([edge_index[1], loop_idx])
    e_tot = e + n
    step = CHUNK * NBUF
    per_tile = -(-e_tot // (NW * step)) * step
    e_pad = per_tile * NW
    src_all = jnp.pad(src_all, (0, e_pad - e_tot)).astype(jnp.int32)
    dst_all = jnp.pad(dst_all, (0, e_pad - e_tot),
                      constant_values=n).astype(jnp.int32)
    nchunks = per_tile // CHUNK
    src_flat = src_all.reshape(NW, per_tile)
    dst_flat = dst_all.reshape(NW, per_tile)
    dst2 = dst_all.reshape(NW, nchunks, CHUNK)

    mesh = plsc.VectorSubcoreMesh(core_axis_name="c", subcore_axis_name="s")
    rec, den = pl.kernel(
        functools.partial(_sc_weights_kernel, nchunks),
        out_type=[
            jax.ShapeDtypeStruct((NW, nchunks * 2 * CHUNK), jnp.int32),
            jax.ShapeDtypeStruct((NW, N_PAD), jnp.float32),
        ],
        mesh=mesh,
        compiler_params=pltpu.CompilerParams(needs_layout_passes=False),
        scratch_types=[
            pltpu.VMEM((per_tile,), jnp.int32),
            pltpu.VMEM((per_tile,), jnp.int32),
            pltpu.VMEM((N_PAD,), jnp.float32),
            pltpu.VMEM((N_PAD,), jnp.float32),
            pltpu.VMEM((N_PAD,), jnp.float32),
            pltpu.VMEM((nchunks * 2 * CHUNK,), jnp.int32),
            pltpu.VMEM((LANES,), jnp.float32),
        ],
    )(src_flat, dst_flat, alpha_s, alpha_d, mvec)

    acc = pl.kernel(
        functools.partial(_sc_scatter_kernel, nchunks),
        out_type=jax.ShapeDtypeStruct((NC, N_PAD, D), jnp.float32),
        mesh=mesh,
        compiler_params=pltpu.CompilerParams(needs_layout_passes=False),
        scratch_types=[
            pltpu.VMEM((nchunks, CHUNK), jnp.int32),
            [pltpu.VMEM((2, CHUNK), jnp.int32) for _ in range(NBUF)],
            [pltpu.VMEM((CHUNK, D), jnp.float32) for _ in range(NBUF)],
            pltpu.VMEM_SHARED((N_PAD, D), jnp.float32),
            [pltpu.SemaphoreType.DMA for _ in range(NBUF)],
            [pltpu.SemaphoreType.DMA for _ in range(NBUF)],
            [pltpu.SemaphoreType.DMA for _ in range(NBUF)],
        ],
    )(rec.reshape(NW, nchunks, 2, CHUNK), dst2, h)

    bsum = (b_conv + bias).reshape(1, D)
    pa = prelu_a.reshape(1, 1)
    fblk = 128
    out = pl.pallas_call(
        _finish_kernel,
        grid=(N_PAD // fblk,),
        in_specs=[
            pl.BlockSpec((NC, fblk, D), lambda i: (0, i, 0)),
            pl.BlockSpec((NW, fblk), lambda i: (0, i)),
            pl.BlockSpec((1, D), lambda i: (0, 0)),
            pl.BlockSpec((1, 1), lambda i: (0, 0)),
        ],
        out_specs=pl.BlockSpec((fblk, D), lambda i: (i, 0)),
        out_shape=jax.ShapeDtypeStruct((N_PAD, D), jnp.float32),
    )(acc, den, bsum, pa)
    return out[:n]


# speedup vs baseline: 33.4198x; 1.0208x over previous
import functools

import jax
import jax.numpy as jnp
from jax import lax
from jax.experimental import pallas as pl
from jax.experimental.pallas import tpu as pltpu
from jax.experimental.pallas import tpu_sc as plsc

N = 10000
D = 128
NC = 2
NS = 16
NW = NC * NS
LANES = 16

CHUNK = 64
NBUF = 3
N_PAD = 10112


def _dense_proj_kernel(seq_ref, wfc_ref, wgat_ref, a2_ref, h_ref, asd_ref,
                       pmax_ref):
    wc = jnp.dot(wfc_ref[...], wgat_ref[...], preferred_element_type=jnp.float32)
    h = jnp.dot(seq_ref[...], wc, preferred_element_type=jnp.float32)
    asd = jnp.dot(h, a2_ref[...], preferred_element_type=jnp.float32)
    h_ref[...] = h
    asd_ref[...] = asd
    m = jnp.max(asd, axis=0, keepdims=True)
    i = pl.program_id(0)

    @pl.when(i == 0)
    def _():
        pmax_ref[...] = m

    @pl.when(i > 0)
    def _():
        pmax_ref[...] = jnp.maximum(pmax_ref[...], m)


def _finish_kernel(acc_ref, den_ref, bsum_ref, pa_ref, out_ref):
    s = acc_ref[0] + acc_ref[1]
    d = jnp.sum(den_ref[...], axis=0)
    o = s / d[:, None] + bsum_ref[...]
    a = pa_ref[0, 0]
    out_ref[...] = jnp.where(o >= 0, o, a * o)


def _sc_weights_kernel(nchunks,
                       src_hbm, dst_hbm, as_hbm, ad_hbm, m_hbm,
                       rec_out, den_out,
                       src_v, dst_v, as_v, ad_v, den_v, rec_t, m_v):
    cid = lax.axis_index("c")
    sid = lax.axis_index("s")
    wid = cid * NS + sid

    pltpu.sync_copy(src_hbm.at[wid], src_v)
    pltpu.sync_copy(dst_hbm.at[wid], dst_v)
    pltpu.sync_copy(as_hbm, as_v)
    pltpu.sync_copy(ad_hbm, ad_v)
    pltpu.sync_copy(m_hbm, m_v)
    mvec = m_v[...]

    def zden(i, _):
        den_v[pl.ds(i * LANES, LANES)] = jnp.zeros((LANES,), jnp.float32)
        return 0
    lax.fori_loop(0, N_PAD // LANES, zden, 0)

    def chunk(c, _):
        ebase = c * CHUNK
        rbase = c * (2 * CHUNK)
        for pos in range(CHUNK // LANES):
            sl = pl.ds(ebase + pos * LANES, LANES)
            sidx = src_v[sl]
            didx = dst_v[sl]
            a_s = plsc.load_gather(as_v, [sidx])
            a_d = plsc.load_gather(ad_v, [didx])
            e = a_s + a_d
            e = jnp.where(e > 0, e, 0.2 * e)
            w = jnp.exp(e - mvec)
            plsc.addupdate_scatter(den_v, [didx], w)
            rec_t[pl.ds(rbase + pos * LANES, LANES)] = sidx
            rec_t[pl.ds(rbase + CHUNK + pos * LANES, LANES)] = (
                plsc.bitcast(w, jnp.int32))
        return 0
    lax.fori_loop(0, nchunks, chunk, 0)

    pltpu.sync_copy(rec_t, rec_out.at[wid])
    pltpu.sync_copy(den_v, den_out.at[wid])


def _sc_scatter_kernel(nchunks,
                       rec_hbm, dst2_hbm, h_hbm,
                       acc_out,
                       dst2_v, pw, rows, acc_sh, sem_f, sem_g, sem_s):
    cid = lax.axis_index("c")
    sid = lax.axis_index("s")
    wid = cid * NS + sid
    rows_per_tile = N_PAD // NS

    pltpu.sync_copy(dst2_hbm.at[wid], dst2_v)

    def zrow(i, _):
        for j in range(D // LANES):
            rows[0][i, pl.ds(j * LANES, LANES)] = jnp.zeros((LANES,),
                                                            jnp.float32)
        return 0
    lax.fori_loop(0, CHUNK, zrow, 0)
    base0 = sid * rows_per_tile
    nfull = rows_per_tile // CHUNK
    zcps = []
    for k in range(nfull):
        zcps.append(pltpu.async_copy(
            rows[0], acc_sh.at[pl.ds(base0 + k * CHUNK, CHUNK)], sem_s[0]))
    rem = rows_per_tile - nfull * CHUNK
    if rem:
        zcps.append(pltpu.async_copy(
            rows[0].at[pl.ds(0, rem)],
            acc_sh.at[pl.ds(base0 + nfull * CHUNK, rem)], sem_s[0]))
    for cp in zcps:
        cp.wait()
    plsc.subcore_barrier()

    def pwfetch(c, b):
        return pltpu.async_copy(rec_hbm.at[wid, c], pw[b], sem_f[b])

    def gather(c, b):
        return pltpu.async_copy(h_hbm.at[pw[b].at[0]], rows[b], sem_g[b])

    def scale(b):
        def body(h2, _):
            i = h2 * 2
            one = jnp.full((LANES,), 1, jnp.int32)
            wsi0 = plsc.load_gather(pw[b], [one,
                                            jnp.full((LANES,), i, jnp.int32)])
            wsi1 = plsc.load_gather(pw[b], [one,
                                            jnp.full((LANES,), i + 1,
                                                     jnp.int32)])
            ws0 = plsc.bitcast(wsi0, jnp.float32)
            ws1 = plsc.bitcast(wsi1, jnp.float32)
            for j in range(D // LANES):
                sl = pl.ds(j * LANES, LANES)
                rows[b][i, sl] = rows[b][i, sl] * ws0
            for j in range(D // LANES):
                sl = pl.ds(j * LANES, LANES)
                rows[b][i + 1, sl] = rows[b][i + 1, sl] * ws1
            return 0
        lax.fori_loop(0, CHUNK // 2, body, 0)

    for b in range(NBUF):
        pwfetch(b, b)
    pltpu.make_async_copy(rec_hbm.at[wid, 0], pw[0], sem_f[0]).wait()
    gather(0, 0)
    pltpu.make_async_copy(rec_hbm.at[wid, 1], pw[1], sem_f[1]).wait()
    gather(1, 1)

    def step(it, _):
        for k in range(NBUF):
            c = it * NBUF + k
            kp = (k + NBUF - 1) % NBUF
            kn = (k + 2) % NBUF
            pltpu.make_async_copy(h_hbm.at[pw[k].at[0]], rows[k],
                                  sem_g[k]).wait()
            scale(k)
            cp_s = pltpu.async_copy(rows[k], acc_sh.at[dst2_v.at[c]],
                                    sem_s[k], add=True)

            @pl.when(c + NBUF < nchunks)
            def _():
                pwfetch(c + NBUF, k)

            @pl.when(c > 0)
            def _():
                pltpu.make_async_copy(
                    rows[kp], acc_sh.at[dst2_v.at[c - 1]], sem_s[kp]).wait()

            @pl.when(c + 2 < nchunks)
            def _():
                pltpu.make_async_copy(rec_hbm.at[wid, c + 2], pw[kn],
                                      sem_f[kn]).wait()
                gather(c + 2, kn)
            return_val = cp_s
        return 0

    lax.fori_loop(0, nchunks // NBUF, step, 0)
    pltpu.make_async_copy(rows[(nchunks - 1) % NBUF],
                          acc_sh.at[dst2_v.at[nchunks - 1]],
                          sem_s[(nchunks - 1) % NBUF]).wait()
    plsc.subcore_barrier()

    stripe = pl.ds(sid * rows_per_tile, rows_per_tile)
    pltpu.sync_copy(acc_sh.at[stripe], acc_out.at[cid, stripe])


def kernel(seq, edge_index, W_fc, W_gat, a_src, a_dst, b_conv, bias, prelu_a):
    n = seq.shape[0]
    e = edge_index.shape[1]

    blk = 2000
    grid = n // blk
    a2 = jnp.stack([a_src, a_dst], axis=1)
    h, asd, pmax = pl.pallas_call(
        _dense_proj_kernel,
        grid=(grid,),
        in_specs=[
            pl.BlockSpec((blk, D), lambda i: (i, 0)),
            pl.BlockSpec((D, D), lambda i: (0, 0)),
            pl.BlockSpec((D, D), lambda i: (0, 0)),
            pl.BlockSpec((D, 2), lambda i: (0, 0)),
        ],
        out_specs=[
            pl.BlockSpec((blk, D), lambda i: (i, 0)),
            pl.BlockSpec((blk, 2), lambda i: (i, 0)),
            pl.BlockSpec((1, 2), lambda i: (0, 0)),
        ],
        out_shape=[
            jax.ShapeDtypeStruct((n, D), jnp.float32),
            jax.ShapeDtypeStruct((n, 2), jnp.float32),
            jax.ShapeDtypeStruct((1, 2), jnp.float32),
        ],
    )(seq, W_fc, W_gat, a2)

    mtot = pmax[0, 0] + pmax[0, 1]
    mshift = jnp.where(mtot > 0, mtot, 0.2 * mtot)
    mvec = jnp.full((LANES,), mshift, jnp.float32)

    alpha_s = jnp.pad(asd[:, 0], (0, N_PAD - n))
    alpha_d = jnp.pad(asd[:, 1], (0, N_PAD - n))

    loop_idx = jnp.arange(n, dtype=edge_index.dtype)
    src_all = jnp.concatenate([edge_index[0], loop_idx])
    dst_all = jnp.concatenate([edge_index[1], loop_idx])
    e_tot = e + n
    step = CHUNK * NBUF
    per_tile = -(-e_tot // (NW * step)) * step
    e_pad = per_tile * NW
    src_all = jnp.pad(src_all, (0, e_pad - e_tot)).astype(jnp.int32)
    dst_all = jnp.pad(dst_all, (0, e_pad - e_tot),
                      constant_values=n).astype(jnp.int32)
    nchunks = per_tile // CHUNK
    src_flat = src_all.reshape(NW, per_tile)
    dst_flat = dst_all.reshape(NW, per_tile)
    dst2 = dst_all.reshape(NW, nchunks, CHUNK)

    mesh = plsc.VectorSubcoreMesh(core_axis_name="c", subcore_axis_name="s")
    rec, den = pl.kernel(
        functools.partial(_sc_weights_kernel, nchunks),
        out_type=[
            jax.ShapeDtypeStruct((NW, nchunks * 2 * CHUNK), jnp.int32),
            jax.ShapeDtypeStruct((NW, N_PAD), jnp.float32),
        ],
        mesh=mesh,
        compiler_params=pltpu.CompilerParams(needs_layout_passes=False),
        scratch_types=[
            pltpu.VMEM((per_tile,), jnp.int32),
            pltpu.VMEM((per_tile,), jnp.int32),
            pltpu.VMEM((N_PAD,), jnp.float32),
            pltpu.VMEM((N_PAD,), jnp.float32),
            pltpu.VMEM((N_PAD,), jnp.float32),
            pltpu.VMEM((nchunks * 2 * CHUNK,), jnp.int32),
            pltpu.VMEM((LANES,), jnp.float32),
        ],
    )(src_flat, dst_flat, alpha_s, alpha_d, mvec)

    acc = pl.kernel(
        functools.partial(_sc_scatter_kernel, nchunks),
        out_type=jax.ShapeDtypeStruct((NC, N_PAD, D), jnp.float32),
        mesh=mesh,
        compiler_params=pltpu.CompilerParams(needs_layout_passes=False),
        scratch_types=[
            pltpu.VMEM((nchunks, CHUNK), jnp.int32),
            [pltpu.VMEM((2, CHUNK), jnp.int32) for _ in range(NBUF)],
            [pltpu.VMEM((CHUNK, D), jnp.float32) for _ in range(NBUF)],
            pltpu.VMEM_SHARED((N_PAD, D), jnp.float32),
            [pltpu.SemaphoreType.DMA for _ in range(NBUF)],
            [pltpu.SemaphoreType.DMA for _ in range(NBUF)],
            [pltpu.SemaphoreType.DMA for _ in range(NBUF)],
        ],
    )(rec.reshape(NW, nchunks, 2, CHUNK), dst2, h)

    bsum = (b_conv + bias).reshape(1, D)
    pa = prelu_a.reshape(1, 1)
    fblk = 128
    out = pl.pallas_call(
        _finish_kernel,
        grid=(N_PAD // fblk,),
        in_specs=[
            pl.BlockSpec((NC, fblk, D), lambda i: (0, i, 0)),
            pl.BlockSpec((NW, fblk), lambda i: (0, i)),
            pl.BlockSpec((1, D), lambda i: (0, 0)),
            pl.BlockSpec((1, 1), lambda i: (0, 0)),
        ],
        out_specs=pl.BlockSpec((fblk, D), lambda i: (i, 0)),
        out_shape=jax.ShapeDtypeStruct((N_PAD, D), jnp.float32),
    )(acc, den, bsum, pa)
    return out[:n]


# speedup vs baseline: 42.0404x; 1.2579x over previous
import functools

import jax
import jax.numpy as jnp
from jax import lax
from jax.experimental import pallas as pl
from jax.experimental.pallas import tpu as pltpu
from jax.experimental.pallas import tpu_sc as plsc

N = 10000
D = 128
NC = 2
NS = 16
NW = NC * NS
LANES = 16

CHUNK = 80
NBUF = 3
N_PAD = 10112


def _dense_proj_kernel(seq_ref, wfc_ref, wgat_ref, a2_ref, h_ref, asd_ref,
                       pmax_ref):
    wc = jnp.dot(wfc_ref[...], wgat_ref[...], preferred_element_type=jnp.float32)
    h = jnp.dot(seq_ref[...], wc, preferred_element_type=jnp.float32)
    asd = jnp.dot(h, a2_ref[...], preferred_element_type=jnp.float32)
    h_ref[...] = h
    asd_ref[...] = asd
    m = jnp.max(asd, axis=0, keepdims=True)
    i = pl.program_id(0)

    @pl.when(i == 0)
    def _():
        pmax_ref[...] = m

    @pl.when(i > 0)
    def _():
        pmax_ref[...] = jnp.maximum(pmax_ref[...], m)


def _finish_kernel(acc_ref, den_ref, bsum_ref, pa_ref, out_ref):
    s = acc_ref[0] + acc_ref[1]
    d = jnp.sum(den_ref[...], axis=0)
    o = s / d[:, None] + bsum_ref[...]
    a = pa_ref[0, 0]
    out_ref[...] = jnp.where(o >= 0, o, a * o)


def _sc_weights_kernel(nchunks,
                       src_hbm, dst_hbm, as_hbm, ad_hbm, m_hbm,
                       rec_out, den_out,
                       src_v, dst_v, as_v, ad_v, den_v, rec_t, m_v):
    cid = lax.axis_index("c")
    sid = lax.axis_index("s")
    wid = cid * NS + sid

    pltpu.sync_copy(src_hbm.at[wid], src_v)
    pltpu.sync_copy(dst_hbm.at[wid], dst_v)
    pltpu.sync_copy(as_hbm, as_v)
    pltpu.sync_copy(ad_hbm, ad_v)
    pltpu.sync_copy(m_hbm, m_v)
    mvec = m_v[...]

    def zden(i, _):
        den_v[pl.ds(i * LANES, LANES)] = jnp.zeros((LANES,), jnp.float32)
        return 0
    lax.fori_loop(0, N_PAD // LANES, zden, 0)

    def chunk(c, _):
        ebase = c * CHUNK
        rbase = c * (2 * CHUNK)
        for pos in range(CHUNK // LANES):
            sl = pl.ds(ebase + pos * LANES, LANES)
            sidx = src_v[sl]
            didx = dst_v[sl]
            a_s = plsc.load_gather(as_v, [sidx])
            a_d = plsc.load_gather(ad_v, [didx])
            e = a_s + a_d
            e = jnp.where(e > 0, e, 0.2 * e)
            w = jnp.exp(e - mvec)
            plsc.addupdate_scatter(den_v, [didx], w)
            rec_t[pl.ds(rbase + pos * LANES, LANES)] = sidx
            rec_t[pl.ds(rbase + CHUNK + pos * LANES, LANES)] = (
                plsc.bitcast(w, jnp.int32))
        return 0
    lax.fori_loop(0, nchunks, chunk, 0)

    pltpu.sync_copy(rec_t, rec_out.at[wid])
    pltpu.sync_copy(den_v, den_out.at[wid])


def _sc_scatter_kernel(nchunks,
                       rec_hbm, dst2_hbm, h_hbm,
                       acc_out,
                       dst2_v, pw, rows, acc_sh, sem_f, sem_g, sem_s):
    cid = lax.axis_index("c")
    sid = lax.axis_index("s")
    wid = cid * NS + sid
    rows_per_tile = N_PAD // NS

    pltpu.sync_copy(dst2_hbm.at[wid], dst2_v)

    def zrow(i, _):
        for j in range(D // LANES):
            rows[0][i, pl.ds(j * LANES, LANES)] = jnp.zeros((LANES,),
                                                            jnp.float32)
        return 0
    lax.fori_loop(0, CHUNK, zrow, 0)
    base0 = sid * rows_per_tile
    nfull = rows_per_tile // CHUNK
    zcps = []
    for k in range(nfull):
        zcps.append(pltpu.async_copy(
            rows[0], acc_sh.at[pl.ds(base0 + k * CHUNK, CHUNK)], sem_s[0]))
    rem = rows_per_tile - nfull * CHUNK
    if rem:
        zcps.append(pltpu.async_copy(
            rows[0].at[pl.ds(0, rem)],
            acc_sh.at[pl.ds(base0 + nfull * CHUNK, rem)], sem_s[0]))
    for cp in zcps:
        cp.wait()
    plsc.subcore_barrier()

    def pwfetch(c, b):
        return pltpu.async_copy(rec_hbm.at[wid, c], pw[b], sem_f[b])

    def gather(c, b):
        return pltpu.async_copy(h_hbm.at[pw[b].at[0]], rows[b], sem_g[b])

    def scale(b):
        def body(h2, _):
            i = h2 * 2
            one = jnp.full((LANES,), 1, jnp.int32)
            wsi0 = plsc.load_gather(pw[b], [one,
                                            jnp.full((LANES,), i, jnp.int32)])
            wsi1 = plsc.load_gather(pw[b], [one,
                                            jnp.full((LANES,), i + 1,
                                                     jnp.int32)])
            ws0 = plsc.bitcast(wsi0, jnp.float32)
            ws1 = plsc.bitcast(wsi1, jnp.float32)
            for j in range(D // LANES):
                sl = pl.ds(j * LANES, LANES)
                rows[b][i, sl] = rows[b][i, sl] * ws0
            for j in range(D // LANES):
                sl = pl.ds(j * LANES, LANES)
                rows[b][i + 1, sl] = rows[b][i + 1, sl] * ws1
            return 0
        lax.fori_loop(0, CHUNK // 2, body, 0)

    for b in range(NBUF):
        pwfetch(b, b)
    pltpu.make_async_copy(rec_hbm.at[wid, 0], pw[0], sem_f[0]).wait()
    gather(0, 0)
    pltpu.make_async_copy(rec_hbm.at[wid, 1], pw[1], sem_f[1]).wait()
    gather(1, 1)

    def step(it, _):
        for k in range(NBUF):
            c = it * NBUF + k
            kp = (k + NBUF - 1) % NBUF
            kn = (k + 2) % NBUF
            pltpu.make_async_copy(h_hbm.at[pw[k].at[0]], rows[k],
                                  sem_g[k]).wait()
            scale(k)
            cp_s = pltpu.async_copy(rows[k], acc_sh.at[dst2_v.at[c]],
                                    sem_s[k], add=True)

            @pl.when(c + NBUF < nchunks)
            def _():
                pwfetch(c + NBUF, k)

            @pl.when(c > 0)
            def _():
                pltpu.make_async_copy(
                    rows[kp], acc_sh.at[dst2_v.at[c - 1]], sem_s[kp]).wait()

            @pl.when(c + 2 < nchunks)
            def _():
                pltpu.make_async_copy(rec_hbm.at[wid, c + 2], pw[kn],
                                      sem_f[kn]).wait()
                gather(c + 2, kn)
        return 0

    lax.fori_loop(0, nchunks // NBUF, step, 0)
    pltpu.make_async_copy(rows[(nchunks - 1) % NBUF],
                          acc_sh.at[dst2_v.at[nchunks - 1]],
                          sem_s[(nchunks - 1) % NBUF]).wait()
    plsc.subcore_barrier()

    stripe = pl.ds(sid * rows_per_tile, rows_per_tile)
    pltpu.sync_copy(acc_sh.at[stripe], acc_out.at[cid, stripe])


def kernel(seq, edge_index, W_fc, W_gat, a_src, a_dst, b_conv, bias, prelu_a):
    n = seq.shape[0]
    e = edge_index.shape[1]

    blk = 2000
    grid = n // blk
    a2 = jnp.stack([a_src, a_dst], axis=1)
    h, asd, pmax = pl.pallas_call(
        _dense_proj_kernel,
        grid=(grid,),
        in_specs=[
            pl.BlockSpec((blk, D), lambda i: (i, 0)),
            pl.BlockSpec((D, D), lambda i: (0, 0)),
            pl.BlockSpec((D, D), lambda i: (0, 0)),
            pl.BlockSpec((D, 2), lambda i: (0, 0)),
        ],
        out_specs=[
            pl.BlockSpec((blk, D), lambda i: (i, 0)),
            pl.BlockSpec((blk, 2), lambda i: (i, 0)),
            pl.BlockSpec((1, 2), lambda i: (0, 0)),
        ],
        out_shape=[
            jax.ShapeDtypeStruct((n, D), jnp.float32),
            jax.ShapeDtypeStruct((n, 2), jnp.float32),
            jax.ShapeDtypeStruct((1, 2), jnp.float32),
        ],
    )(seq, W_fc, W_gat, a2)

    mtot = pmax[0, 0] + pmax[0, 1]
    mshift = jnp.where(mtot > 0, mtot, 0.2 * mtot)
    mvec = jnp.full((LANES,), mshift, jnp.float32)

    alpha_s = jnp.pad(asd[:, 0], (0, N_PAD - n))
    alpha_d = jnp.pad(asd[:, 1], (0, N_PAD - n))

    loop_idx = jnp.arange(n, dtype=edge_index.dtype)
    src_all = jnp.concatenate([edge_index[0], loop_idx])
    dst_all = jnp.concatenate([edge_index[1], loop_idx])
    e_tot = e + n
    step = CHUNK * NBUF
    per_tile = -(-e_tot // (NW * step)) * step
    e_pad = per_tile * NW
    src_all = jnp.pad(src_all, (0, e_pad - e_tot)).astype(jnp.int32)
    dst_all = jnp.pad(dst_all, (0, e_pad - e_tot),
                      constant_values=n).astype(jnp.int32)
    nchunks = per_tile // CHUNK
    src_flat = src_all.reshape(NW, per_tile)
    dst_flat = dst_all.reshape(NW, per_tile)
    dst2 = dst_all.reshape(NW, nchunks, CHUNK)

    mesh = plsc.VectorSubcoreMesh(core_axis_name="c", subcore_axis_name="s")
    rec, den = pl.kernel(
        functools.partial(_sc_weights_kernel, nchunks),
        out_type=[
            jax.ShapeDtypeStruct((NW, nchunks * 2 * CHUNK), jnp.int32),
            jax.ShapeDtypeStruct((NW, N_PAD), jnp.float32),
        ],
        mesh=mesh,
        compiler_params=pltpu.CompilerParams(needs_layout_passes=False),
        scratch_types=[
            pltpu.VMEM((per_tile,), jnp.int32),
            pltpu.VMEM((per_tile,), jnp.int32),
            pltpu.VMEM((N_PAD,), jnp.float32),
            pltpu.VMEM((N_PAD,), jnp.float32),
            pltpu.VMEM((N_PAD,), jnp.float32),
            pltpu.VMEM((nchunks * 2 * CHUNK,), jnp.int32),
            pltpu.VMEM((LANES,), jnp.float32),
        ],
    )(src_flat, dst_flat, alpha_s, alpha_d, mvec)

    acc = pl.kernel(
        functools.partial(_sc_scatter_kernel, nchunks),
        out_type=jax.ShapeDtypeStruct((NC, N_PAD, D), jnp.float32),
        mesh=mesh,
        compiler_params=pltpu.CompilerParams(needs_layout_passes=False),
        scratch_types=[
            pltpu.VMEM((nchunks, CHUNK), jnp.int32),
            [pltpu.VMEM((2, CHUNK), jnp.int32) for _ in range(NBUF)],
            [pltpu.VMEM((CHUNK, D), jnp.float32) for _ in range(NBUF)],
            pltpu.VMEM_SHARED((N_PAD, D), jnp.float32),
            [pltpu.SemaphoreType.DMA for _ in range(NBUF)],
            [pltpu.SemaphoreType.DMA for _ in range(NBUF)],
            [pltpu.SemaphoreType.DMA for _ in range(NBUF)],
        ],
    )(rec.reshape(NW, nchunks, 2, CHUNK), dst2, h)

    bsum = (b_conv + bias).reshape(1, D)
    pa = prelu_a.reshape(1, 1)
    fblk = 128
    out = pl.pallas_call(
        _finish_kernel,
        grid=(N_PAD // fblk,),
        in_specs=[
            pl.BlockSpec((NC, fblk, D), lambda i: (0, i, 0)),
            pl.BlockSpec((NW, fblk), lambda i: (0, i)),
            pl.BlockSpec((1, D), lambda i: (0, 0)),
            pl.BlockSpec((1, 1), lambda i: (0, 0)),
        ],
        out_specs=pl.BlockSpec((fblk, D), lambda i: (i, 0)),
        out_shape=jax.ShapeDtypeStruct((N_PAD, D), jnp.float32),
    )(acc, den, bsum, pa)
    return out[:n]
